# baseline (speedup 1.0000x reference)
import functools

import jax
import jax.numpy as jnp
import numpy as np
from jax import lax
from jax.experimental import pallas as pl
from jax.experimental.pallas import tpu as pltpu
from jax.experimental.pallas import tpu_sc as plsc

VOCAB_NUM = 100000
SEG_NUM = 2
MAX_SEQ_LEN = 2048
D_MODEL = 768
BATCH = 4

NC = 2
NS = 16
NW = NC * NS
B_TOTAL = BATCH * MAX_SEQ_LEN
B_PER_W = B_TOTAL // NW
W_PER_B = MAX_SEQ_LEN // B_PER_W
CHUNK = 32
NBUF = 2
N_CHUNKS = B_PER_W // CHUNK
LANES = 16
GROUPS = D_MODEL // LANES


def _pos_enc_table():
    pos_ids = np.arange(0, MAX_SEQ_LEN, 1, dtype=np.float32)[:, None]
    div_term = np.power(
        10000.0, np.arange(0, D_MODEL, 2, dtype=np.float32) / D_MODEL)
    pe = np.zeros((MAX_SEQ_LEN, D_MODEL), dtype=np.float32)
    pe[:, ::2] = np.sin(pos_ids / div_term)
    pe[:, 1::2] = np.cos(pos_ids / div_term)
    return pe


_POS_ENC = _pos_enc_table()


@functools.partial(
    pl.kernel,
    out_type=jax.ShapeDtypeStruct((BATCH, MAX_SEQ_LEN, D_MODEL), jnp.float32),
    mesh=plsc.VectorSubcoreMesh(core_axis_name="c", subcore_axis_name="s"),
    scratch_types=[
        pltpu.VMEM((B_PER_W,), jnp.int32),
        pltpu.VMEM((B_PER_W + LANES,), jnp.int32),
        pltpu.VMEM((SEG_NUM, D_MODEL), jnp.float32),
        [pltpu.VMEM((CHUNK, D_MODEL), jnp.float32)] * NBUF,
        [pltpu.VMEM((CHUNK, D_MODEL), jnp.float32)] * NBUF,
        pltpu.SemaphoreType.DMA,
        [pltpu.SemaphoreType.DMA] * NBUF,
        [pltpu.SemaphoreType.DMA] * NBUF,
        [pltpu.SemaphoreType.DMA] * NBUF,
    ],
)
def _emb_kernel(ids_hbm, segs_hbm, tok_hbm, segt_hbm, pos_hbm, out_hbm,
                idx_t, idx_s, seg_v, buf_a, buf_p, sem_i, sem_a,
                sem_p, sem_o):
    wid = lax.axis_index("s") * NC + lax.axis_index("c")
    b = wid // W_PER_B
    pos_base = (wid % W_PER_B) * B_PER_W

    cp_t = pltpu.async_copy(ids_hbm.at[b, pl.ds(pos_base, B_PER_W)], idx_t,
                            sem_i)
    cp_s = pltpu.async_copy(segs_hbm.at[b, pl.ds(pos_base, B_PER_W)],
                            idx_s.at[pl.ds(0, B_PER_W)], sem_i)
    cp_g = pltpu.async_copy(segt_hbm, seg_v, sem_i)
    cp_t.wait()
    cp_s.wait()
    cp_g.wait()

    def issue(k, s):
        cp_a = pltpu.async_copy(tok_hbm.at[idx_t.at[pl.ds(k * CHUNK, CHUNK)]],
                                buf_a[s], sem_a[s])
        cp_p = pltpu.async_copy(
            pos_hbm.at[pl.ds(pos_base + k * CHUNK, CHUNK)], buf_p[s],
            sem_p[s])
        return cp_a, cp_p

    gathers = [None] * NBUF
    stores = [None] * NBUF
    for k in range(NBUF - 1):
        gathers[k] = issue(k, k)
    for k in range(N_CHUNKS):
        s = k % NBUF
        if k + NBUF - 1 < N_CHUNKS:
            n = (k + NBUF - 1) % NBUF
            if stores[n] is not None:
                stores[n].wait()
            gathers[n] = issue(k + NBUF - 1, n)
        cp_a, cp_p = gathers[s]
        cp_a.wait()
        cp_p.wait()

        def add_row(r, _, s=s, k=k):
            sr = idx_s[pl.ds(k * CHUNK + r, LANES)][0]
            for g in range(GROUPS):
                sl = pl.ds(g * LANES, LANES)
                plsc.addupdate(buf_a[s].at[r, sl], buf_p[s][r, sl])
                plsc.addupdate(buf_a[s].at[r, sl], seg_v[sr, sl])
            return 0

        lax.fori_loop(0, CHUNK, add_row, 0)
        stores[s] = pltpu.async_copy(
            buf_a[s], out_hbm.at[b, pl.ds(pos_base + k * CHUNK, CHUNK)],
            sem_o[s])
    for st in stores:
        if st is not None:
            st.wait()


def kernel(input_ids, seg_ids, masks, token_table, segment_table):
    del masks
    return _emb_kernel(input_ids.astype(jnp.int32),
                       seg_ids.astype(jnp.int32), token_table,
                       segment_table, jnp.asarray(_POS_ENC))
